# baseline (speedup 1.0000x reference)
import jax
import jax.numpy as jnp
from jax import lax
from jax.experimental import pallas as pl
from jax.experimental.pallas import tpu as pltpu
from jax.experimental.pallas import tpu_sc as plsc

N_NODES = 100000
N_EDGES = 3200000

NC, NS, L = 2, 16, 16
NW = NC * NS
C = 2000
CHUNKS = N_EDGES // C
CPW = CHUNKS // NW
IDX_T = 125
N_T = 2 * C // IDX_T
GROUPS = C // L

_HALF_PI = 1.5707963267948966
_PI = 3.141592653589793
_QPI = 0.7853981633974483
_TAN_PI_8 = 0.4142135623730951


def _rsqrt(x):
    i = plsc.bitcast(x, jnp.int32)
    i = 0x5F3759DF - lax.shift_right_arithmetic(i, 1)
    y = plsc.bitcast(i, jnp.float32)
    y = y * (1.5 - 0.5 * x * y * y)
    y = y * (1.5 - 0.5 * x * y * y)
    y = y * (1.5 - 0.5 * x * y * y)
    return y


def _atan01(z):
    red = z > _TAN_PI_8
    zr = jnp.where(red, (z - 1.0) / (z + 1.0), z)
    base = jnp.where(red, _QPI, 0.0)
    s = zr * zr
    p = (((8.05374449538e-2 * s - 1.38776856032e-1) * s + 1.99777106478e-1) * s
         - 3.33329491539e-1) * s * zr + zr
    return base + p


def _cross(ax, ay, az, bx, by, bz):
    return ay * bz - az * by, az * bx - ax * bz, ax * by - ay * bx


def _qmul(a, b):
    ax, ay, az, aw = a
    bx, by, bz, bw = b
    return (aw * bx + ax * bw + ay * bz - az * by,
            aw * by - ax * bz + ay * bw + az * bx,
            aw * bz + ax * by - ay * bx + az * bw,
            aw * bw - ax * bx - ay * by - az * bz)


def _qrot(q, v):
    qx, qy, qz, qw = q
    vx, vy, vz = v
    tx, ty, tz = _cross(qx, qy, qz, vx, vy, vz)
    tx, ty, tz = tx + tx, ty + ty, tz + tz
    cx, cy, cz = _cross(qx, qy, qz, tx, ty, tz)
    return vx + qw * tx + cx, vy + qw * ty + cy, vz + qw * tz + cz


def _edge_math(pose, n1, n2):
    tp = pose[0:3]
    qpi = (-pose[3], -pose[4], -pose[5], pose[6])
    t1 = n1[0:3]
    q1i = (-n1[3], -n1[4], -n1[5], n1[6])
    t2 = n2[0:3]
    q2 = (n2[3], n2[4], n2[5], n2[6])

    qA = _qmul(qpi, q1i)
    qx, qy, qz, qw = _qmul(qA, q2)
    d = (t2[0] - t1[0], t2[1] - t1[1], t2[2] - t1[2])
    ra = _qrot(qpi, tp)
    rb = _qrot(qA, d)
    tx, ty, tz = rb[0] - ra[0], rb[1] - ra[1], rb[2] - ra[2]

    ss = qx * qx + qy * qy + qz * qz
    n = ss * _rsqrt(jnp.maximum(ss, 1e-35))
    aw = jnp.abs(qw)
    mx = jnp.maximum(n, aw)
    mn = jnp.minimum(n, aw)
    a = _atan01(mn / mx)
    a = jnp.where(n > aw, _HALF_PI - a, a)
    a = jnp.where(qw < 0.0, _PI - a, a)
    theta = a + a
    n_small = n < 1e-8
    k_num = jnp.where(n_small, 2.0, theta)
    k_den = jnp.where(n_small, jnp.where(aw < 1e-8, 1.0, qw), n)
    k = k_num / k_den
    px, py, pz = k * qx, k * qy, k * qz
    small = theta < 1e-4
    th_safe = jnp.where(small, 1.0, theta)
    nb_safe = jnp.where(small, 1.0, n)
    b = jnp.where(small, 1.0 / 12.0,
                  (1.0 - theta * qw / (nb_safe + nb_safe)) / (th_safe * th_safe))
    c1x, c1y, c1z = _cross(px, py, pz, tx, ty, tz)
    c2x, c2y, c2z = _cross(px, py, pz, c1x, c1y, c1z)
    rx = tx - 0.5 * c1x + b * c2x
    ry = ty - 0.5 * c1y + b * c2y
    rz = tz - 0.5 * c1z + b * c2z
    return rx, ry, rz, px, py, pz


def _body(edges_hbm, poses_hbm, nodes_hbm, out_hbm,
          idx_v, rows_v, pose_v, out_v, sems):
    wid = lax.axis_index("s") * NC + lax.axis_index("c")
    iota = lax.iota(jnp.int32, L)
    first = wid * CPW

    def copies(c, buf):
        cps = []
        for h in range(2):
            for j in range(C // IDX_T):
                cps.append(pltpu.make_async_copy(
                    nodes_hbm.at[idx_v.at[buf].at[h].at[j]],
                    rows_v.at[buf].at[pl.ds((h * C) + j * IDX_T, IDX_T)],
                    sems.at[buf]))
        cps.append(pltpu.make_async_copy(
            poses_hbm.at[pl.ds(c * C, C), pl.ds(0, 7)], pose_v.at[buf],
            sems.at[buf]))
        return cps

    def fire(c, buf):
        pltpu.sync_copy(edges_hbm.at[:, c], idx_v.at[buf])
        for cp in copies(c, buf):
            cp.start()

    def drain(c, buf):
        for cp in copies(c, buf):
            cp.wait()

    def compute(buf):
        rows = rows_v.at[buf]
        poses = pose_v.at[buf]

        def group(g, carry):
            ids = g * L + iota
            pose = tuple(plsc.load_gather(poses, [ids, jnp.full((L,), f, jnp.int32)])
                         for f in range(7))
            n1 = tuple(plsc.load_gather(rows, [ids, jnp.full((L,), f, jnp.int32)])
                       for f in range(7))
            n2 = tuple(plsc.load_gather(rows, [ids + C, jnp.full((L,), f, jnp.int32)])
                       for f in range(7))
            res = _edge_math(pose, n1, n2)
            for f in range(6):
                plsc.store_scatter(out_v, [ids, jnp.full((L,), f, jnp.int32)],
                                   res[f])
            return carry

        lax.fori_loop(0, GROUPS, group, 0)

    def store_out(c):
        pltpu.sync_copy(out_v, out_hbm.at[pl.ds(c * C, C)])

    fire(first, 0)

    def step(i, carry):
        c0 = first + 2 * i
        fire(c0 + 1, 1)
        drain(c0, 0)
        compute(0)
        store_out(c0)

        @pl.when(i + 1 < CPW // 2)
        def _():
            fire(c0 + 2, 0)

        drain(c0 + 1, 1)
        compute(1)
        store_out(c0 + 1)
        return carry

    lax.fori_loop(0, CPW // 2, step, 0)


_RBLK = 16000
_RNB = N_EDGES // _RBLK


def _poses_to_aos(pt):
    def body(in_ref, out_ref):
        out_ref[:, :7] = in_ref[...].T

    return pl.pallas_call(
        body,
        grid=(_RNB,),
        in_specs=[pl.BlockSpec((7, _RBLK), lambda c: (0, c))],
        out_specs=pl.BlockSpec((_RBLK, 8), lambda c: (c, 0)),
        out_shape=jax.ShapeDtypeStruct((N_EDGES, 8), jnp.float32),
    )(pt)


def _aos_to_out(aos):
    def body(in_ref, out_ref):
        out_ref[...] = in_ref[:, :6].T

    return pl.pallas_call(
        body,
        grid=(_RNB,),
        in_specs=[pl.BlockSpec((_RBLK, 8), lambda c: (c, 0))],
        out_specs=pl.BlockSpec((6, _RBLK), lambda c: (0, c)),
        out_shape=jax.ShapeDtypeStruct((6, N_EDGES), jnp.float32),
    )(aos)


def kernel(edges, poses, nodes):
    edges_t = edges.T.reshape(2, CHUNKS, C // IDX_T, IDX_T)
    poses_aos = _poses_to_aos(poses.T)
    nodes_p = jnp.pad(nodes, ((0, 0), (0, 1)))
    mesh = plsc.VectorSubcoreMesh(core_axis_name="c", subcore_axis_name="s",
                                  num_cores=NC, num_subcores=NS)
    out_aos = pl.kernel(
        _body,
        out_type=jax.ShapeDtypeStruct((N_EDGES, 8), jnp.float32),
        mesh=mesh,
        compiler_params=pltpu.CompilerParams(needs_layout_passes=False,
                                             use_tc_tiling_on_sc=False),
        scratch_types=[
            pltpu.VMEM((2, 2, C // IDX_T, IDX_T), jnp.int32),
            pltpu.VMEM((2, 2 * C, 8), jnp.float32),
            pltpu.VMEM((2, C, 7), jnp.float32),
            pltpu.VMEM((C, 8), jnp.float32),
            pltpu.SemaphoreType.DMA((2,)),
        ],
    )(edges_t, poses_aos, nodes_p)
    return _aos_to_out(out_aos).T


# speedup vs baseline: 3.2918x; 3.2918x over previous
import jax
import jax.numpy as jnp
from jax import lax
from jax.experimental import pallas as pl
from jax.experimental.pallas import tpu as pltpu
from jax.experimental.pallas import tpu_sc as plsc

N_NODES = 100000
N_EDGES = 3200000

NC, NS, L = 2, 16, 16
NW = NC * NS
C = 2000
CHUNKS = N_EDGES // C
CPW = CHUNKS // NW
IDX_T = 125
N_T = 2 * C // IDX_T
GROUPS = C // L

_HALF_PI = 1.5707963267948966
_PI = 3.141592653589793
_QPI = 0.7853981633974483
_TAN_PI_8 = 0.4142135623730951


def _rsqrt(x):
    i = plsc.bitcast(x, jnp.int32)
    i = 0x5F3759DF - lax.shift_right_arithmetic(i, 1)
    y = plsc.bitcast(i, jnp.float32)
    y = y * (1.5 - 0.5 * x * y * y)
    y = y * (1.5 - 0.5 * x * y * y)
    y = y * (1.5 - 0.5 * x * y * y)
    return y


def _atan01(z):
    red = z > _TAN_PI_8
    zr = jnp.where(red, (z - 1.0) / (z + 1.0), z)
    base = jnp.where(red, _QPI, 0.0)
    s = zr * zr
    p = (((8.05374449538e-2 * s - 1.38776856032e-1) * s + 1.99777106478e-1) * s
         - 3.33329491539e-1) * s * zr + zr
    return base + p


def _cross(ax, ay, az, bx, by, bz):
    return ay * bz - az * by, az * bx - ax * bz, ax * by - ay * bx


def _qmul(a, b):
    ax, ay, az, aw = a
    bx, by, bz, bw = b
    return (aw * bx + ax * bw + ay * bz - az * by,
            aw * by - ax * bz + ay * bw + az * bx,
            aw * bz + ax * by - ay * bx + az * bw,
            aw * bw - ax * bx - ay * by - az * bz)


def _qrot(q, v):
    qx, qy, qz, qw = q
    vx, vy, vz = v
    tx, ty, tz = _cross(qx, qy, qz, vx, vy, vz)
    tx, ty, tz = tx + tx, ty + ty, tz + tz
    cx, cy, cz = _cross(qx, qy, qz, tx, ty, tz)
    return vx + qw * tx + cx, vy + qw * ty + cy, vz + qw * tz + cz


def _edge_math(pose, n1, n2):
    tp = pose[0:3]
    qpi = (-pose[3], -pose[4], -pose[5], pose[6])
    t1 = n1[0:3]
    q1i = (-n1[3], -n1[4], -n1[5], n1[6])
    t2 = n2[0:3]
    q2 = (n2[3], n2[4], n2[5], n2[6])

    qA = _qmul(qpi, q1i)
    qx, qy, qz, qw = _qmul(qA, q2)
    d = (t2[0] - t1[0], t2[1] - t1[1], t2[2] - t1[2])
    ra = _qrot(qpi, tp)
    rb = _qrot(qA, d)
    tx, ty, tz = rb[0] - ra[0], rb[1] - ra[1], rb[2] - ra[2]

    ss = qx * qx + qy * qy + qz * qz
    n = ss * _rsqrt(jnp.maximum(ss, 1e-35))
    aw = jnp.abs(qw)
    mx = jnp.maximum(n, aw)
    mn = jnp.minimum(n, aw)
    a = _atan01(mn / mx)
    a = jnp.where(n > aw, _HALF_PI - a, a)
    a = jnp.where(qw < 0.0, _PI - a, a)
    theta = a + a
    n_small = n < 1e-8
    k_num = jnp.where(n_small, 2.0, theta)
    k_den = jnp.where(n_small, jnp.where(aw < 1e-8, 1.0, qw), n)
    k = k_num / k_den
    px, py, pz = k * qx, k * qy, k * qz
    small = theta < 1e-4
    th_safe = jnp.where(small, 1.0, theta)
    nb_safe = jnp.where(small, 1.0, n)
    b = jnp.where(small, 1.0 / 12.0,
                  (1.0 - theta * qw / (nb_safe + nb_safe)) / (th_safe * th_safe))
    c1x, c1y, c1z = _cross(px, py, pz, tx, ty, tz)
    c2x, c2y, c2z = _cross(px, py, pz, c1x, c1y, c1z)
    rx = tx - 0.5 * c1x + b * c2x
    ry = ty - 0.5 * c1y + b * c2y
    rz = tz - 0.5 * c1z + b * c2z
    return rx, ry, rz, px, py, pz


def _body(edges_hbm, poses_hbm, nodes_hbm, out_hbm,
          idx_v, rows_v, pose_v, out_v, sems):
    wid = lax.axis_index("s") * NC + lax.axis_index("c")
    iota = lax.iota(jnp.int32, L)
    first = wid * CPW

    def copies(c, buf):
        rb = lax.div(c, _RBLK // C)
        off = lax.rem(c, _RBLK // C) * C
        cps = []
        for h in range(2):
            for j in range(C // IDX_T):
                cps.append(pltpu.make_async_copy(
                    nodes_hbm.at[idx_v.at[buf].at[h].at[j]],
                    rows_v.at[buf].at[pl.ds((h * C) + j * IDX_T, IDX_T)],
                    sems.at[buf]))
        for f in range(7):
            cps.append(pltpu.make_async_copy(
                poses_hbm.at[rb].at[f].at[pl.ds(off, C)],
                pose_v.at[buf].at[f], sems.at[buf]))
        return cps

    def fire(c, buf):
        pltpu.sync_copy(edges_hbm.at[:, c], idx_v.at[buf])
        for cp in copies(c, buf):
            cp.start()

    def drain(c, buf):
        for cp in copies(c, buf):
            cp.wait()

    def compute(buf):
        rows = rows_v.at[buf]
        poses = pose_v.at[buf]

        def group(g, carry):
            s = pl.ds(g * L, L)
            ids = g * L + iota
            pose = tuple(poses[f, s] for f in range(7))
            n1 = tuple(plsc.load_gather(rows, [ids, jnp.full((L,), f, jnp.int32)])
                       for f in range(7))
            n2 = tuple(plsc.load_gather(rows, [ids + C, jnp.full((L,), f, jnp.int32)])
                       for f in range(7))
            res = _edge_math(pose, n1, n2)
            for f in range(6):
                out_v[f, s] = res[f]
            return carry

        lax.fori_loop(0, GROUPS, group, 0, unroll=5)

    def store_out(c):
        rb = lax.div(c, _RBLK // C)
        off = lax.rem(c, _RBLK // C) * C
        for f in range(6):
            pltpu.sync_copy(out_v.at[f], out_hbm.at[rb].at[f].at[pl.ds(off, C)])

    fire(first, 0)

    def step(i, carry):
        c0 = first + 2 * i
        fire(c0 + 1, 1)
        drain(c0, 0)
        compute(0)
        store_out(c0)

        @pl.when(i + 1 < CPW // 2)
        def _():
            fire(c0 + 2, 0)

        drain(c0 + 1, 1)
        compute(1)
        store_out(c0 + 1)
        return carry

    lax.fori_loop(0, CPW // 2, step, 0)


_RBLK = 128000
_RNB = N_EDGES // _RBLK


def _poses_to_soa(pt):
    def body(in_ref, out_ref):
        out_ref[0] = in_ref[...]

    return pl.pallas_call(
        body,
        grid=(_RNB,),
        in_specs=[pl.BlockSpec((7, _RBLK), lambda c: (0, c))],
        out_specs=pl.BlockSpec((1, 7, _RBLK), lambda c: (c, 0, 0)),
        out_shape=jax.ShapeDtypeStruct((_RNB, 7, _RBLK), jnp.float32),
    )(pt)


def _soa_to_out(soa):
    def body(in_ref, out_ref):
        out_ref[...] = in_ref[0]

    return pl.pallas_call(
        body,
        grid=(_RNB,),
        in_specs=[pl.BlockSpec((1, 6, _RBLK), lambda c: (c, 0, 0))],
        out_specs=pl.BlockSpec((6, _RBLK), lambda c: (0, c)),
        out_shape=jax.ShapeDtypeStruct((6, N_EDGES), jnp.float32),
    )(soa)


def kernel(edges, poses, nodes):
    edges_t = edges.T.reshape(2, CHUNKS, C // IDX_T, IDX_T)
    poses_soa = _poses_to_soa(poses.T)
    nodes_p = jnp.pad(nodes, ((0, 0), (0, 1)))
    mesh = plsc.VectorSubcoreMesh(core_axis_name="c", subcore_axis_name="s",
                                  num_cores=NC, num_subcores=NS)
    out_soa = pl.kernel(
        _body,
        out_type=jax.ShapeDtypeStruct((_RNB, 6, _RBLK), jnp.float32),
        mesh=mesh,
        compiler_params=pltpu.CompilerParams(needs_layout_passes=False,
                                             use_tc_tiling_on_sc=False),
        scratch_types=[
            pltpu.VMEM((2, 2, C // IDX_T, IDX_T), jnp.int32),
            pltpu.VMEM((2, 2 * C, 8), jnp.float32),
            pltpu.VMEM((2, 7, C), jnp.float32),
            pltpu.VMEM((6, C), jnp.float32),
            pltpu.SemaphoreType.DMA((2,)),
        ],
    )(edges_t, poses_soa, nodes_p)
    return _soa_to_out(out_soa).T
